# baseline (speedup 1.0000x reference)
import functools

import jax
import jax.numpy as jnp
from jax import lax
from jax.experimental import pallas as pl
from jax.experimental.pallas import tpu as pltpu
from jax.experimental.pallas import tpu_sc as plsc

N_ROWS = 1000000
N_COLS = 64
N_UPD = 16384

NW = 32
RB = 31248
REM = N_ROWS - NW * RB
L = 16
NVEC = N_UPD // L
CP = 496
NCH = RB // CP
GCAP = 48

_mesh = plsc.VectorSubcoreMesh(core_axis_name="c", subcore_axis_name="s")


@functools.partial(
    pl.kernel,
    out_type=jax.ShapeDtypeStruct((N_ROWS, N_COLS), jnp.float32),
    mesh=_mesh,
    scratch_types=[
        pltpu.VMEM((N_UPD,), jnp.int32),
        pltpu.VMEM((RB + REM,), jnp.int32),
        pltpu.VMEM((GCAP + L,), jnp.int32),
        pltpu.VMEM((GCAP + L,), jnp.int32),
        pltpu.VMEM((L, 2 * N_COLS), jnp.float32),
        pltpu.VMEM((CP, N_COLS), jnp.float32),
        pltpu.SemaphoreType.DMA,
        pltpu.SemaphoreType.DMA,
        pltpu.SemaphoreType.DMA,
    ],
    compiler_params=pltpu.CompilerParams(
        needs_layout_passes=False, use_tc_tiling_on_sc=True),
)
def _sc_scatter(data_hbm, idx_hbm, upd2_hbm, out_hbm,
                idxv, tagv, gidx, grel, rowbuf, cpbuf,
                sem_f, sem_p, sem_g):
    c = lax.axis_index("c")
    s = lax.axis_index("s")
    wid = c * 16 + s
    base = wid * RB
    hi = base + RB + jnp.where(wid == NW - 1, REM, 0)
    lane = lax.iota(jnp.int32, L)
    neg1 = jnp.full((L,), -1, jnp.int32)

    def init_tag(v, carry):
        tagv[pl.ds(v * L, L)] = neg1
        return carry

    lax.fori_loop(0, (RB + REM) // L, init_tag, 0)

    pltpu.sync_copy(idx_hbm, idxv)

    def p1(k, carry):
        iv = idxv[pl.ds(k * L, L)]
        inr = (iv >= base) & (iv < hi)
        rel = jnp.where(inr, iv - base, 0)
        plsc.store_scatter(tagv, [rel], lane + k * L, mask=inr)
        return carry

    lax.fori_loop(0, NVEC, p1, 0)

    zero = jnp.zeros((L,), jnp.int32)

    def process_chunk(crel, nrows):
        pltpu.async_copy(data_hbm.at[pl.ds(base + crel, nrows)],
                         cpbuf.at[pl.ds(0, nrows)], sem_f).wait()

        for b in range((GCAP + L) // L):
            gidx[pl.ds(b * L, L)] = zero
            grel[pl.ds(b * L, L)] = zero

        def scan(v, cnt):
            tv = tagv[pl.ds(crel + v * L, L)]
            valid = tv >= 0
            vi = valid.astype(jnp.int32)
            pos = cnt + plsc.cumsum(vi) - 1
            posc = jnp.where(valid, pos, 0)
            plsc.store_scatter(gidx, [posc], tv, mask=valid)
            plsc.store_scatter(grel, [posc], lane + v * L, mask=valid)
            return cnt + jnp.sum(vi)

        cnt = lax.fori_loop(0, nrows // L, scan, 0)

        for b in range(GCAP // L):
            @pl.when(cnt > b * L)
            def _batch(b=b):
                gv = gidx[pl.ds(b * L, L)]
                relv = grel[pl.ds(b * L, L)]
                pltpu.async_copy(upd2_hbm.at[gv >> 1], rowbuf, sem_g).wait()
                for g in range(L):
                    @pl.when(b * L + g < cnt)
                    def _merge(g=g, gv=gv, relv=relv):
                        h = (gv[g] & 1) * N_COLS
                        r = relv[g]
                        for k in range(N_COLS // L):
                            cpbuf[r, pl.ds(k * L, L)] = (
                                rowbuf[g, pl.ds(h + k * L, L)])

        pltpu.async_copy(cpbuf.at[pl.ds(0, nrows)],
                         out_hbm.at[pl.ds(base + crel, nrows)], sem_p).wait()

    def chunk_body(q, carry):
        process_chunk(q * CP, CP)
        return carry

    lax.fori_loop(0, NCH, chunk_body, 0)

    @pl.when(wid == NW - 1)
    def _tail():
        process_chunk(jnp.int32(RB), REM)


def kernel(data, indices, updates):
    idx = indices.reshape(-1)
    upd2 = updates.reshape(N_UPD // 2, 2 * N_COLS)
    return _sc_scatter(data, idx, upd2)
